# Initial kernel scaffold; baseline (speedup 1.0000x reference)
import jax
import jax.numpy as jnp
from jax.experimental import pallas as pl


def kernel(x, edge_index, edge_attr, params):
    raise NotImplementedError("write your pallas kernel here")


# speedup vs baseline: 18.8692x; 18.8692x over previous
import functools

import jax
import jax.numpy as jnp
from jax import lax
from jax.experimental import pallas as pl
from jax.experimental.pallas import tpu as pltpu
from jax.experimental.pallas import tpu_sc as plsc

N = 10000
E = 320000
D = 128
NC, NS = 2, 16
NW = NC * NS
EPW = E // NW
C = 80
NCHUNK = EPW // C
WB = 640

def _mesh():
    return plsc.VectorSubcoreMesh(core_axis_name="c", subcore_axis_name="s",
                                  num_cores=NC, num_subcores=NS)


def _tc_edge_in(ea, We, be, Ae):
    BE = 4000
    H = Ae.shape[1]
    K = ea.shape[1]

    def body(ea_ref, we_ref, be_ref, ae_ref, w_ref, ee_ref):
        w = jnp.dot(ea_ref[...], we_ref[...], preferred_element_type=jnp.float32)
        w = jnp.maximum(w + be_ref[...], 0.0)
        w_ref[...] = w
        ee_ref[...] = jnp.dot(w, ae_ref[...], preferred_element_type=jnp.float32)

    return pl.pallas_call(
        body,
        grid=(E // BE,),
        in_specs=[pl.BlockSpec((BE, K), lambda i: (i, 0)),
                  pl.BlockSpec((K, D), lambda i: (0, 0)),
                  pl.BlockSpec((1, D), lambda i: (0, 0)),
                  pl.BlockSpec((D, H), lambda i: (0, 0))],
        out_specs=[pl.BlockSpec((BE, D), lambda i: (i, 0)),
                   pl.BlockSpec((BE, H), lambda i: (i, 0))],
        out_shape=[jax.ShapeDtypeStruct((E, D), jnp.float32),
                   jax.ShapeDtypeStruct((E, H), jnp.float32)],
    )(ea, We, be, Ae)


def _tc_edge_update(w, u, Wm, bm, Ae):
    BE = 4000
    H = Ae.shape[1]

    def body(w_ref, u_ref, wm_ref, bm_ref, ae_ref, wo_ref, ee_ref):
        wn = jnp.dot(w_ref[...], wm_ref[...], preferred_element_type=jnp.float32)
        wn = jnp.maximum(wn + u_ref[...] + bm_ref[...], 0.0)
        wo_ref[...] = wn
        ee_ref[...] = jnp.dot(wn, ae_ref[...], preferred_element_type=jnp.float32)

    return pl.pallas_call(
        body,
        grid=(E // BE,),
        in_specs=[pl.BlockSpec((BE, D), lambda i: (i, 0)),
                  pl.BlockSpec((BE, D), lambda i: (i, 0)),
                  pl.BlockSpec((D, D), lambda i: (0, 0)),
                  pl.BlockSpec((1, D), lambda i: (0, 0)),
                  pl.BlockSpec((D, H), lambda i: (0, 0))],
        out_specs=[pl.BlockSpec((BE, D), lambda i: (i, 0)),
                   pl.BlockSpec((BE, H), lambda i: (i, 0))],
        out_shape=[jax.ShapeDtypeStruct((E, D), jnp.float32),
                   jax.ShapeDtypeStruct((E, H), jnp.float32)],
    )(w, u, Wm, bm, Ae)


def _tc_node_in(x, Wn, bn, fcW, Alr):
    BN = 2000
    H2 = Alr.shape[1]
    K = x.shape[1]

    def body(x_ref, wn_ref, bn_ref, fc_ref, alr_ref, ft_ref, eler_ref):
        h = jnp.dot(x_ref[...], wn_ref[...], preferred_element_type=jnp.float32)
        h = jnp.maximum(h + bn_ref[...], 0.0)
        ft_ref[...] = jnp.dot(h, fc_ref[...], preferred_element_type=jnp.float32)
        eler_ref[...] = jnp.dot(h, alr_ref[...], preferred_element_type=jnp.float32)

    return pl.pallas_call(
        body,
        grid=(N // BN,),
        in_specs=[pl.BlockSpec((BN, K), lambda i: (i, 0)),
                  pl.BlockSpec((K, D), lambda i: (0, 0)),
                  pl.BlockSpec((1, D), lambda i: (0, 0)),
                  pl.BlockSpec((D, D), lambda i: (0, 0)),
                  pl.BlockSpec((D, H2), lambda i: (0, 0))],
        out_specs=[pl.BlockSpec((BN, D), lambda i: (i, 0)),
                   pl.BlockSpec((BN, H2), lambda i: (i, 0))],
        out_shape=[jax.ShapeDtypeStruct((N, D), jnp.float32),
                   jax.ShapeDtypeStruct((N, H2), jnp.float32)],
    )(x, Wn, bn, fcW, Alr)


def _tc_node_mid(msgP, denP, H, bias, S, fcW, Alr, Wemb):
    BN = 2000
    H2 = Alr.shape[1]

    def body(m_ref, d_ref, b_ref, s_ref, fc_ref, alr_ref, we_ref,
             ft_ref, eler_ref, hw_ref):
        m = m_ref[0] + m_ref[1]
        den = (d_ref[0] + d_ref[1])[:, :H]
        inv = 1.0 / (den + 1e-9)
        invf = jnp.dot(inv, s_ref[...], preferred_element_type=jnp.float32)
        h = jnp.maximum(m * invf + b_ref[...], 0.0)
        ft_ref[...] = jnp.dot(h, fc_ref[...], preferred_element_type=jnp.float32)
        eler_ref[...] = jnp.dot(h, alr_ref[...], preferred_element_type=jnp.float32)
        hw_ref[...] = jnp.dot(h, we_ref[...], preferred_element_type=jnp.float32)

    return pl.pallas_call(
        body,
        grid=(N // BN,),
        in_specs=[pl.BlockSpec((2, BN, D), lambda i: (0, i, 0)),
                  pl.BlockSpec((2, BN, 16), lambda i: (0, i, 0)),
                  pl.BlockSpec((1, D), lambda i: (0, 0)),
                  pl.BlockSpec((H, D), lambda i: (0, 0)),
                  pl.BlockSpec((D, D), lambda i: (0, 0)),
                  pl.BlockSpec((D, H2), lambda i: (0, 0)),
                  pl.BlockSpec((D, D), lambda i: (0, 0))],
        out_specs=[pl.BlockSpec((BN, D), lambda i: (i, 0)),
                   pl.BlockSpec((BN, H2), lambda i: (i, 0)),
                   pl.BlockSpec((BN, D), lambda i: (i, 0))],
        out_shape=[jax.ShapeDtypeStruct((N, D), jnp.float32),
                   jax.ShapeDtypeStruct((N, H2), jnp.float32),
                   jax.ShapeDtypeStruct((N, D), jnp.float32)],
    )(msgP, denP, bias, S, fcW, Alr, Wemb)


def _tc_node_last(msgP, denP, H, bias, S):
    BN = 2000

    def body(m_ref, d_ref, b_ref, s_ref, o_ref):
        m = m_ref[0] + m_ref[1]
        den = (d_ref[0] + d_ref[1])[:, :H]
        inv = 1.0 / (den + 1e-9)
        invf = jnp.dot(inv, s_ref[...], preferred_element_type=jnp.float32)
        o_ref[...] = m * invf + b_ref[...]

    return pl.pallas_call(
        body,
        grid=(N // BN,),
        in_specs=[pl.BlockSpec((2, BN, D), lambda i: (0, i, 0)),
                  pl.BlockSpec((2, BN, 16), lambda i: (0, i, 0)),
                  pl.BlockSpec((1, D), lambda i: (0, 0)),
                  pl.BlockSpec((H, D), lambda i: (0, 0))],
        out_specs=pl.BlockSpec((BN, D), lambda i: (i, 0)),
        out_shape=jax.ShapeDtypeStruct((N, D), jnp.float32),
    )(msgP, denP, bias, S)


_SC_PARAMS = pltpu.CompilerParams(needs_layout_passes=False,
                                  use_tc_tiling_on_sc=False)


@functools.lru_cache(maxsize=None)
def _sc_msg(H):
    scratch = [
        pltpu.VMEM((C, D), jnp.float32),
        pltpu.VMEM((C,), jnp.int32),
        pltpu.VMEM((C,), jnp.int32),
        pltpu.VMEM((C, 16), jnp.float32),
        pltpu.VMEM((C, 16), jnp.float32),
        pltpu.VMEM((C * H,), jnp.float32),
        pltpu.VMEM((C, 16), jnp.float32),
        pltpu.VMEM_SHARED((N, D), jnp.float32),
        pltpu.VMEM_SHARED((N, 16), jnp.float32),
        pltpu.SemaphoreType.DMA,
        pltpu.SemaphoreType.DMA,
        pltpu.SemaphoreType.DMA,
    ]

    @functools.partial(
        pl.kernel,
        out_type=[jax.ShapeDtypeStruct((NC, N, D), jnp.float32),
                  jax.ShapeDtypeStruct((NC * N, 16), jnp.float32)],
        mesh=_mesh(),
        scratch_types=scratch,
        compiler_params=_SC_PARAMS)
    def msg_kernel(ft_hbm, eler_hbm, ee_hbm, src_hbm, dst_hbm,
                   msg_hbm, den_hbm,
                   rows, sidx, didx, elb, erb, eev, pbuf,
                   msg_s, den_s, sem, sem2, sem3):
        cid = lax.axis_index("c")
        sid = lax.axis_index("s")
        wid = sid * NC + cid
        zero16 = jnp.zeros((16,), jnp.float32)
        iot = lax.iota(jnp.int32, 16)

        def _zp(i, _):
            pbuf[i, :] = zero16
            return 0
        lax.fori_loop(0, C, _zp, 0)

        def _zr(i, _):
            for j in range(8):
                rows[i, pl.ds(j * 16, 16)] = zero16
            return 0
        lax.fori_loop(0, C, _zr, 0)

        start = pl.multiple_of(sid * WB, WB)
        for t in range(5):
            pltpu.sync_copy(rows, msg_s.at[pl.ds(start + t * 80, 80)])
            pltpu.sync_copy(pbuf, den_s.at[pl.ds(start + t * 80, 80)])

        @pl.when(sid < NS - 1)
        def _zfull():
            for t in range(5, 8):
                pltpu.sync_copy(rows, msg_s.at[pl.ds(start + t * 80, 80)])
                pltpu.sync_copy(pbuf, den_s.at[pl.ds(start + t * 80, 80)])
        plsc.subcore_barrier()

        def chunk(k, _):
            base = pl.multiple_of((wid * EPW + k * C) * 1, 80)
            pltpu.sync_copy(src_hbm.at[pl.ds(base, C)], sidx)
            pltpu.sync_copy(dst_hbm.at[pl.ds(base, C)], didx)
            gat = pltpu.async_copy(ft_hbm.at[sidx], rows, sem)
            gel = pltpu.async_copy(eler_hbm.at[sidx], elb, sem2)
            ger = pltpu.async_copy(eler_hbm.at[didx], erb, sem3)
            pltpu.sync_copy(ee_hbm.at[pl.ds(base * H, C * H)], eev)
            gel.wait()
            ger.wait()
            for g in range(C // 16):
                r16 = g * 16 + iot
                for h in range(H):
                    elv = plsc.load_gather(
                        elb, [r16, jnp.full((16,), h, jnp.int32)])
                    erv = plsc.load_gather(
                        erb, [r16, jnp.full((16,), 8 + h, jnp.int32)])
                    if H == 1:
                        eevv = eev[pl.ds(g * 16, 16)]
                    else:
                        eevv = plsc.load_gather(eev, [r16 * H + h])
                    t = elv + erv + eevv
                    t = jnp.maximum(t, 0.2 * t)
                    p = jnp.exp(t)
                    plsc.store_scatter(
                        pbuf, [r16, jnp.full((16,), h, jnp.int32)], p)
            gat.wait()

            def erow(e2, _):
                prow = pbuf[e2, :]
                for j in range(8):
                    ps = prow[0] if H == 1 else prow[j // 2]
                    sl = pl.ds(j * 16, 16)
                    rows[e2, sl] = rows[e2, sl] * ps
                return 0
            lax.fori_loop(0, C, erow, 0)

            pltpu.sync_copy(rows, msg_s.at[didx], add=True)
            pltpu.sync_copy(pbuf, den_s.at[didx], add=True)
            return 0
        lax.fori_loop(0, NCHUNK, chunk, 0)
        plsc.subcore_barrier()

        dstart = pl.multiple_of(cid * N, N) + start

        @pl.when(sid < NS - 1)
        def _wbfull():
            pltpu.sync_copy(msg_s.at[pl.ds(start, WB)],
                            msg_hbm.at[cid, pl.ds(start, WB)])
            pltpu.sync_copy(den_s.at[pl.ds(start, WB)],
                            den_hbm.at[pl.ds(dstart, WB)])

        @pl.when(sid == NS - 1)
        def _wbtail():
            pltpu.sync_copy(msg_s.at[pl.ds(start, 400)],
                            msg_hbm.at[cid, pl.ds(start, 400)])
            pltpu.sync_copy(den_s.at[pl.ds(start, 400)],
                            den_hbm.at[pl.ds(dstart, 400)])

    return msg_kernel


@functools.lru_cache(maxsize=None)
def _sc_u():
    scratch = [
        pltpu.VMEM((C,), jnp.int32),
        pltpu.VMEM((C,), jnp.int32),
        pltpu.VMEM((C, D), jnp.float32),
        pltpu.VMEM((C, D), jnp.float32),
        pltpu.SemaphoreType.DMA,
    ]

    @functools.partial(
        pl.kernel,
        out_type=jax.ShapeDtypeStruct((E, D), jnp.float32),
        mesh=_mesh(),
        scratch_types=scratch,
        compiler_params=_SC_PARAMS)
    def u_kernel(hw_hbm, src_hbm, dst_hbm, u_hbm, sidx, didx, bufa, bufb, sem):
        cid = lax.axis_index("c")
        sid = lax.axis_index("s")
        wid = sid * NC + cid

        def chunk(k, _):
            base = wid * EPW + k * C
            pltpu.sync_copy(src_hbm.at[pl.ds(base, C)], sidx)
            pltpu.sync_copy(dst_hbm.at[pl.ds(base, C)], didx)
            d1 = pltpu.async_copy(hw_hbm.at[sidx], bufa, sem)
            d1.wait()
            d2 = pltpu.async_copy(hw_hbm.at[didx], bufb, sem)
            d2.wait()

            def erow(e2, _):
                for j in range(8):
                    sl = pl.ds(j * 16, 16)
                    bufa[e2, sl] = bufa[e2, sl] + bufb[e2, sl]
                return 0
            lax.fori_loop(0, C, erow, 0)
            pltpu.sync_copy(bufa, u_hbm.at[pl.ds(base, C)])
            return 0
        lax.fori_loop(0, NCHUNK, chunk, 0)

    return u_kernel


def kernel(x, edge_index, edge_attr, params):
    src = edge_index[0]
    dst = edge_index[1]
    layers = params['layers']
    cfgs = [(4, 32), (4, 32), (1, 128)]

    Alr, Ae, S, bias2d = [], [], [], []
    for i, (H, F) in enumerate(cfgs):
        p = layers[i]
        Al = (p['fc_W'].reshape(D, H, F) * p['attn_l'][None]).sum(-1)
        Ar = (p['fc_W'].reshape(D, H, F) * p['attn_r'][None]).sum(-1)
        z = jnp.zeros((D, 8 - H), jnp.float32)
        Alr.append(jnp.concatenate([Al, z, Ar, z], axis=1))
        Ae.append((p['fc_edge_W'].reshape(D, H, F) * p['attn_e'][None]).sum(-1))
        S.append(jnp.repeat(jnp.eye(H, dtype=jnp.float32), F, axis=1))
        bias2d.append(p['bias'].reshape(1, D))
    emb = params['edge_emb']

    ft0, eler0 = _tc_node_in(x, params['node_W'], params['node_b'].reshape(1, D),
                             layers[0]['fc_W'], Alr[0])
    w0, ee0 = _tc_edge_in(edge_attr, params['edge_W'],
                          params['edge_b'].reshape(1, D), Ae[0])
    msg0, den0 = _sc_msg(4)(ft0, eler0, ee0.reshape(-1), src, dst)
    ft1, eler1, hW1 = _tc_node_mid(msg0, den0.reshape(NC, N, 16), 4, bias2d[0],
                                   S[0], layers[1]['fc_W'], Alr[1], emb[0]['W'])

    u0 = _sc_u()(hW1, src, dst)
    w1, ee1 = _tc_edge_update(w0, u0, emb[0]['W'], emb[0]['b'].reshape(1, D),
                              Ae[1])

    msg1, den1 = _sc_msg(4)(ft1, eler1, ee1.reshape(-1), src, dst)
    ft2, eler2, hW2 = _tc_node_mid(msg1, den1.reshape(NC, N, 16), 4, bias2d[1],
                                   S[1], layers[2]['fc_W'], Alr[2], emb[1]['W'])

    u1 = _sc_u()(hW2, src, dst)
    w2, ee2 = _tc_edge_update(w1, u1, emb[1]['W'], emb[1]['b'].reshape(1, D),
                              Ae[2])

    msg2, den2 = _sc_msg(1)(ft2, eler2, ee2.reshape(-1), src, dst)
    return _tc_node_last(msg2, den2.reshape(NC, N, 16), 1, bias2d[2], S[2])


# speedup vs baseline: 25.1476x; 1.3327x over previous
import functools

import jax
import jax.numpy as jnp
from jax import lax
from jax.experimental import pallas as pl
from jax.experimental.pallas import tpu as pltpu
from jax.experimental.pallas import tpu_sc as plsc

N = 10000
E = 320000
D = 128
NC, NS = 2, 16
NW = NC * NS
EPW = E // NW
C = 80
NCHUNK = EPW // C
WB = 640

def _mesh():
    return plsc.VectorSubcoreMesh(core_axis_name="c", subcore_axis_name="s",
                                  num_cores=NC, num_subcores=NS)


def _tc_edge_in(ea, We, be, Ae):
    BE = 4000
    H = Ae.shape[1]
    K = ea.shape[1]

    def body(ea_ref, we_ref, be_ref, ae_ref, w_ref, ee_ref):
        w = jnp.dot(ea_ref[...], we_ref[...], preferred_element_type=jnp.float32)
        w = jnp.maximum(w + be_ref[...], 0.0)
        w_ref[...] = w
        ee_ref[...] = jnp.dot(w, ae_ref[...], preferred_element_type=jnp.float32)

    return pl.pallas_call(
        body,
        grid=(E // BE,),
        in_specs=[pl.BlockSpec((BE, K), lambda i: (i, 0)),
                  pl.BlockSpec((K, D), lambda i: (0, 0)),
                  pl.BlockSpec((1, D), lambda i: (0, 0)),
                  pl.BlockSpec((D, H), lambda i: (0, 0))],
        out_specs=[pl.BlockSpec((BE, D), lambda i: (i, 0)),
                   pl.BlockSpec((BE, H), lambda i: (i, 0))],
        out_shape=[jax.ShapeDtypeStruct((E, D), jnp.float32),
                   jax.ShapeDtypeStruct((E, H), jnp.float32)],
    )(ea, We, be, Ae)


def _tc_edge_update(w, u, Wm, bm, Ae):
    BE = 4000
    H = Ae.shape[1]

    def body(w_ref, u_ref, wm_ref, bm_ref, ae_ref, wo_ref, ee_ref):
        wn = jnp.dot(w_ref[...], wm_ref[...], preferred_element_type=jnp.float32)
        wn = jnp.maximum(wn + u_ref[...] + bm_ref[...], 0.0)
        wo_ref[...] = wn
        ee_ref[...] = jnp.dot(wn, ae_ref[...], preferred_element_type=jnp.float32)

    return pl.pallas_call(
        body,
        grid=(E // BE,),
        in_specs=[pl.BlockSpec((BE, D), lambda i: (i, 0)),
                  pl.BlockSpec((BE, D), lambda i: (i, 0)),
                  pl.BlockSpec((D, D), lambda i: (0, 0)),
                  pl.BlockSpec((1, D), lambda i: (0, 0)),
                  pl.BlockSpec((D, H), lambda i: (0, 0))],
        out_specs=[pl.BlockSpec((BE, D), lambda i: (i, 0)),
                   pl.BlockSpec((BE, H), lambda i: (i, 0))],
        out_shape=[jax.ShapeDtypeStruct((E, D), jnp.float32),
                   jax.ShapeDtypeStruct((E, H), jnp.float32)],
    )(w, u, Wm, bm, Ae)


def _tc_node_in(x, Wn, bn, fcW, Alr):
    BN = 2000
    H2 = Alr.shape[1]
    K = x.shape[1]

    def body(x_ref, wn_ref, bn_ref, fc_ref, alr_ref, ft_ref, eler_ref):
        h = jnp.dot(x_ref[...], wn_ref[...], preferred_element_type=jnp.float32)
        h = jnp.maximum(h + bn_ref[...], 0.0)
        ft_ref[...] = jnp.dot(h, fc_ref[...], preferred_element_type=jnp.float32)
        eler_ref[...] = jnp.dot(h, alr_ref[...], preferred_element_type=jnp.float32)

    return pl.pallas_call(
        body,
        grid=(N // BN,),
        in_specs=[pl.BlockSpec((BN, K), lambda i: (i, 0)),
                  pl.BlockSpec((K, D), lambda i: (0, 0)),
                  pl.BlockSpec((1, D), lambda i: (0, 0)),
                  pl.BlockSpec((D, D), lambda i: (0, 0)),
                  pl.BlockSpec((D, H2), lambda i: (0, 0))],
        out_specs=[pl.BlockSpec((BN, D), lambda i: (i, 0)),
                   pl.BlockSpec((BN, H2), lambda i: (i, 0))],
        out_shape=[jax.ShapeDtypeStruct((N, D), jnp.float32),
                   jax.ShapeDtypeStruct((N, H2), jnp.float32)],
    )(x, Wn, bn, fcW, Alr)


def _tc_node_mid(msgP, denP, H, bias, S, fcW, Alr, Wemb):
    BN = 2000
    H2 = Alr.shape[1]

    def body(m_ref, d_ref, b_ref, s_ref, fc_ref, alr_ref, we_ref,
             ft_ref, eler_ref, hw_ref):
        m = m_ref[0] + m_ref[1]
        den = (d_ref[0] + d_ref[1])[:, :H]
        inv = 1.0 / (den + 1e-9)
        invf = jnp.dot(inv, s_ref[...], preferred_element_type=jnp.float32)
        h = jnp.maximum(m * invf + b_ref[...], 0.0)
        ft_ref[...] = jnp.dot(h, fc_ref[...], preferred_element_type=jnp.float32)
        eler_ref[...] = jnp.dot(h, alr_ref[...], preferred_element_type=jnp.float32)
        hw_ref[...] = jnp.dot(h, we_ref[...], preferred_element_type=jnp.float32)

    return pl.pallas_call(
        body,
        grid=(N // BN,),
        in_specs=[pl.BlockSpec((2, BN, D), lambda i: (0, i, 0)),
                  pl.BlockSpec((2, BN, 16), lambda i: (0, i, 0)),
                  pl.BlockSpec((1, D), lambda i: (0, 0)),
                  pl.BlockSpec((H, D), lambda i: (0, 0)),
                  pl.BlockSpec((D, D), lambda i: (0, 0)),
                  pl.BlockSpec((D, H2), lambda i: (0, 0)),
                  pl.BlockSpec((D, D), lambda i: (0, 0))],
        out_specs=[pl.BlockSpec((BN, D), lambda i: (i, 0)),
                   pl.BlockSpec((BN, H2), lambda i: (i, 0)),
                   pl.BlockSpec((BN, D), lambda i: (i, 0))],
        out_shape=[jax.ShapeDtypeStruct((N, D), jnp.float32),
                   jax.ShapeDtypeStruct((N, H2), jnp.float32),
                   jax.ShapeDtypeStruct((N, D), jnp.float32)],
    )(msgP, denP, bias, S, fcW, Alr, Wemb)


def _tc_node_last(msgP, denP, H, bias, S):
    BN = 2000

    def body(m_ref, d_ref, b_ref, s_ref, o_ref):
        m = m_ref[0] + m_ref[1]
        den = (d_ref[0] + d_ref[1])[:, :H]
        inv = 1.0 / (den + 1e-9)
        invf = jnp.dot(inv, s_ref[...], preferred_element_type=jnp.float32)
        o_ref[...] = m * invf + b_ref[...]

    return pl.pallas_call(
        body,
        grid=(N // BN,),
        in_specs=[pl.BlockSpec((2, BN, D), lambda i: (0, i, 0)),
                  pl.BlockSpec((2, BN, 16), lambda i: (0, i, 0)),
                  pl.BlockSpec((1, D), lambda i: (0, 0)),
                  pl.BlockSpec((H, D), lambda i: (0, 0))],
        out_specs=pl.BlockSpec((BN, D), lambda i: (i, 0)),
        out_shape=jax.ShapeDtypeStruct((N, D), jnp.float32),
    )(msgP, denP, bias, S)


_SC_PARAMS = pltpu.CompilerParams(needs_layout_passes=False,
                                  use_tc_tiling_on_sc=False)


@functools.lru_cache(maxsize=None)
def _sc_msg(H):
    scratch = [
        pltpu.VMEM((C, D), jnp.float32),
        pltpu.VMEM((C, D), jnp.float32),
        pltpu.VMEM((C,), jnp.int32),
        pltpu.VMEM((C,), jnp.int32),
        pltpu.VMEM((C,), jnp.int32),
        pltpu.VMEM((C,), jnp.int32),
        pltpu.VMEM((C, 16), jnp.float32),
        pltpu.VMEM((C, 16), jnp.float32),
        pltpu.VMEM((C, 16), jnp.float32),
        pltpu.VMEM((C, 16), jnp.float32),
        pltpu.VMEM((C * H,), jnp.float32),
        pltpu.VMEM((C * H,), jnp.float32),
        pltpu.VMEM((C, 16), jnp.float32),
        pltpu.VMEM_SHARED((N, D), jnp.float32),
        pltpu.VMEM_SHARED((N, 16), jnp.float32),
    ] + [pltpu.SemaphoreType.DMA] * 8

    @functools.partial(
        pl.kernel,
        out_type=[jax.ShapeDtypeStruct((NC, N, D), jnp.float32),
                  jax.ShapeDtypeStruct((NC * N, 16), jnp.float32)],
        mesh=_mesh(),
        scratch_types=scratch,
        compiler_params=_SC_PARAMS)
    def msg_kernel(ft_hbm, eler_hbm, ee_hbm, src_hbm, dst_hbm,
                   msg_hbm, den_hbm,
                   rows0, rows1, sidx0, sidx1, didx0, didx1,
                   elb0, elb1, erb0, erb1, eev0, eev1, pbuf,
                   msg_s, den_s,
                   smf0, smf1, sml0, sml1, smr0, smr1, sme0, sme1):
        cid = lax.axis_index("c")
        sid = lax.axis_index("s")
        wid = sid * NC + cid
        zero16 = jnp.zeros((16,), jnp.float32)
        iot = lax.iota(jnp.int32, 16)
        rows = (rows0, rows1)
        sidx = (sidx0, sidx1)
        didx = (didx0, didx1)
        elb = (elb0, elb1)
        erb = (erb0, erb1)
        eev = (eev0, eev1)
        smf = (smf0, smf1)
        sml = (sml0, sml1)
        smr = (smr0, smr1)
        sme = (sme0, sme1)

        def _zp(i, _):
            pbuf[i, :] = zero16
            return 0
        lax.fori_loop(0, C, _zp, 0)

        def _zr(i, _):
            for j in range(8):
                rows0[i, pl.ds(j * 16, 16)] = zero16
            return 0
        lax.fori_loop(0, C, _zr, 0)

        start = pl.multiple_of(sid * WB, WB)
        for t in range(5):
            pltpu.sync_copy(rows0, msg_s.at[pl.ds(start + t * 80, 80)])
            pltpu.sync_copy(pbuf, den_s.at[pl.ds(start + t * 80, 80)])

        @pl.when(sid < NS - 1)
        def _zfull():
            for t in range(5, 8):
                pltpu.sync_copy(rows0, msg_s.at[pl.ds(start + t * 80, 80)])
                pltpu.sync_copy(pbuf, den_s.at[pl.ds(start + t * 80, 80)])
        plsc.subcore_barrier()

        def issue(k, P):
            base = pl.multiple_of(wid * EPW + k * C, 80)
            pltpu.sync_copy(src_hbm.at[pl.ds(base, C)], sidx[P])
            pltpu.sync_copy(dst_hbm.at[pl.ds(base, C)], didx[P])
            pltpu.async_copy(ft_hbm.at[sidx[P]], rows[P], smf[P])
            pltpu.async_copy(eler_hbm.at[sidx[P]], elb[P], sml[P])
            pltpu.async_copy(eler_hbm.at[didx[P]], erb[P], smr[P])
            pltpu.async_copy(ee_hbm.at[pl.ds(base * H, C * H)], eev[P], sme[P])

        def process(k, P):
            pltpu.make_async_copy(eler_hbm.at[sidx[P]], elb[P], sml[P]).wait()
            pltpu.make_async_copy(eler_hbm.at[didx[P]], erb[P], smr[P]).wait()
            pltpu.make_async_copy(
                ee_hbm.at[pl.ds(0, C * H)], eev[P], sme[P]).wait()
            for g in range(C // 16):
                r16 = g * 16 + iot
                for h in range(H):
                    elv = plsc.load_gather(
                        elb[P], [r16, jnp.full((16,), h, jnp.int32)])
                    erv = plsc.load_gather(
                        erb[P], [r16, jnp.full((16,), 8 + h, jnp.int32)])
                    if H == 1:
                        eevv = eev[P][pl.ds(g * 16, 16)]
                    else:
                        eevv = plsc.load_gather(eev[P], [r16 * H + h])
                    t = elv + erv + eevv
                    t = jnp.maximum(t, 0.2 * t)
                    p = jnp.exp(t)
                    plsc.store_scatter(
                        pbuf, [r16, jnp.full((16,), h, jnp.int32)], p)
            pltpu.make_async_copy(ft_hbm.at[sidx[P]], rows[P], smf[P]).wait()

            def erow(e2, _):
                prow = pbuf[e2, :]
                for j in range(8):
                    ps = prow[0] if H == 1 else prow[j // 2]
                    sl = pl.ds(j * 16, 16)
                    rows[P][e2, sl] = rows[P][e2, sl] * ps
                return 0
            lax.fori_loop(0, C, erow, 0)

            pltpu.sync_copy(rows[P], msg_s.at[didx[P]], add=True)
            pltpu.sync_copy(pbuf, den_s.at[didx[P]], add=True)

        issue(0, 0)

        def pair(k2, _):
            k = k2 * 2
            issue(k + 1, 1)
            process(k, 0)
            issue(k + 2, 0)
            process(k + 1, 1)
            return 0
        lax.fori_loop(0, (NCHUNK - 1) // 2, pair, 0)
        process(NCHUNK - 1, 0)
        plsc.subcore_barrier()

        dstart = pl.multiple_of(cid * N, N) + start

        @pl.when(sid < NS - 1)
        def _wbfull():
            pltpu.sync_copy(msg_s.at[pl.ds(start, WB)],
                            msg_hbm.at[cid, pl.ds(start, WB)])
            pltpu.sync_copy(den_s.at[pl.ds(start, WB)],
                            den_hbm.at[pl.ds(dstart, WB)])

        @pl.when(sid == NS - 1)
        def _wbtail():
            pltpu.sync_copy(msg_s.at[pl.ds(start, 400)],
                            msg_hbm.at[cid, pl.ds(start, 400)])
            pltpu.sync_copy(den_s.at[pl.ds(start, 400)],
                            den_hbm.at[pl.ds(dstart, 400)])

    return msg_kernel


@functools.lru_cache(maxsize=None)
def _sc_u():
    scratch = [
        pltpu.VMEM((C,), jnp.int32),
        pltpu.VMEM((C,), jnp.int32),
        pltpu.VMEM((C,), jnp.int32),
        pltpu.VMEM((C,), jnp.int32),
        pltpu.VMEM((C, D), jnp.float32),
        pltpu.VMEM((C, D), jnp.float32),
        pltpu.VMEM((C, D), jnp.float32),
        pltpu.VMEM((C, D), jnp.float32),
    ] + [pltpu.SemaphoreType.DMA] * 4

    @functools.partial(
        pl.kernel,
        out_type=jax.ShapeDtypeStruct((E, D), jnp.float32),
        mesh=_mesh(),
        scratch_types=scratch,
        compiler_params=_SC_PARAMS)
    def u_kernel(hw_hbm, src_hbm, dst_hbm, u_hbm,
                 sidx0, sidx1, didx0, didx1, bufa0, bufa1, bufb0, bufb1,
                 sma0, sma1, smb0, smb1):
        cid = lax.axis_index("c")
        sid = lax.axis_index("s")
        wid = sid * NC + cid
        sidx = (sidx0, sidx1)
        didx = (didx0, didx1)
        bufa = (bufa0, bufa1)
        bufb = (bufb0, bufb1)
        sma = (sma0, sma1)
        smb = (smb0, smb1)

        def issue(k, P):
            base = pl.multiple_of(wid * EPW + k * C, 80)
            pltpu.sync_copy(src_hbm.at[pl.ds(base, C)], sidx[P])
            pltpu.sync_copy(dst_hbm.at[pl.ds(base, C)], didx[P])
            pltpu.async_copy(hw_hbm.at[sidx[P]], bufa[P], sma[P])
            pltpu.async_copy(hw_hbm.at[didx[P]], bufb[P], smb[P])

        def process(k, P):
            base = pl.multiple_of(wid * EPW + k * C, 80)
            pltpu.make_async_copy(hw_hbm.at[sidx[P]], bufa[P], sma[P]).wait()
            pltpu.make_async_copy(hw_hbm.at[didx[P]], bufb[P], smb[P]).wait()

            def erow(e2, _):
                for j in range(8):
                    sl = pl.ds(j * 16, 16)
                    bufa[P][e2, sl] = bufa[P][e2, sl] + bufb[P][e2, sl]
                return 0
            lax.fori_loop(0, C, erow, 0)
            pltpu.sync_copy(bufa[P], u_hbm.at[pl.ds(base, C)])

        issue(0, 0)

        def pair(k2, _):
            k = k2 * 2
            issue(k + 1, 1)
            process(k, 0)
            issue(k + 2, 0)
            process(k + 1, 1)
            return 0
        lax.fori_loop(0, (NCHUNK - 1) // 2, pair, 0)
        process(NCHUNK - 1, 0)

    return u_kernel


def kernel(x, edge_index, edge_attr, params):
    src = edge_index[0]
    dst = edge_index[1]
    layers = params['layers']
    cfgs = [(4, 32), (4, 32), (1, 128)]

    Alr, Ae, S, bias2d = [], [], [], []
    for i, (H, F) in enumerate(cfgs):
        p = layers[i]
        Al = (p['fc_W'].reshape(D, H, F) * p['attn_l'][None]).sum(-1)
        Ar = (p['fc_W'].reshape(D, H, F) * p['attn_r'][None]).sum(-1)
        z = jnp.zeros((D, 8 - H), jnp.float32)
        Alr.append(jnp.concatenate([Al, z, Ar, z], axis=1))
        Ae.append((p['fc_edge_W'].reshape(D, H, F) * p['attn_e'][None]).sum(-1))
        S.append(jnp.repeat(jnp.eye(H, dtype=jnp.float32), F, axis=1))
        bias2d.append(p['bias'].reshape(1, D))
    emb = params['edge_emb']

    ft0, eler0 = _tc_node_in(x, params['node_W'], params['node_b'].reshape(1, D),
                             layers[0]['fc_W'], Alr[0])
    w0, ee0 = _tc_edge_in(edge_attr, params['edge_W'],
                          params['edge_b'].reshape(1, D), Ae[0])
    msg0, den0 = _sc_msg(4)(ft0, eler0, ee0.reshape(-1), src, dst)
    ft1, eler1, hW1 = _tc_node_mid(msg0, den0.reshape(NC, N, 16), 4, bias2d[0],
                                   S[0], layers[1]['fc_W'], Alr[1], emb[0]['W'])

    u0 = _sc_u()(hW1, src, dst)
    w1, ee1 = _tc_edge_update(w0, u0, emb[0]['W'], emb[0]['b'].reshape(1, D),
                              Ae[1])

    msg1, den1 = _sc_msg(4)(ft1, eler1, ee1.reshape(-1), src, dst)
    ft2, eler2, hW2 = _tc_node_mid(msg1, den1.reshape(NC, N, 16), 4, bias2d[1],
                                   S[1], layers[2]['fc_W'], Alr[2], emb[1]['W'])

    u1 = _sc_u()(hW2, src, dst)
    w2, ee2 = _tc_edge_update(w1, u1, emb[1]['W'], emb[1]['b'].reshape(1, D),
                              Ae[2])

    msg2, den2 = _sc_msg(1)(ft2, eler2, ee2.reshape(-1), src, dst)
    return _tc_node_last(msg2, den2.reshape(NC, N, 16), 1, bias2d[2], S[2])
